# baseline (speedup 1.0000x reference)
import functools

import jax
import jax.numpy as jnp
import numpy as np
from jax import lax
from jax.experimental import pallas as pl
from jax.experimental.pallas import tpu as pltpu
from jax.experimental.pallas import tpu_sc as plsc

_EPS = float(np.finfo(np.float32).eps)
_K = 8192
_R = 1024
_C = 1024
_NC = 2
_NS = 16
_NW = _NC * _NS
_ROWS_W = _R // _NW
_LANES = 16
_NH = 2
_VECS_ROW = _C // _LANES


def _sc_hist(quant_info):
    mesh = plsc.VectorSubcoreMesh(core_axis_name="c", subcore_axis_name="s")

    @functools.partial(
        pl.kernel,
        out_type=jax.ShapeDtypeStruct((_NW * _NH, _K), jnp.int32),
        mesh=mesh,
        scratch_types=[
            pltpu.VMEM((_ROWS_W, _C), jnp.int32),
            pltpu.VMEM((_K,), jnp.int32),
            pltpu.VMEM((_K,), jnp.int32),
        ],
        compiler_params=pltpu.CompilerParams(
            needs_layout_passes=False, disable_bounds_checks=True),
    )
    def hist_kernel(idx_hbm, out_hbm, idx_v, hist0_v, hist1_v):
        wid = lax.axis_index("s") * _NC + lax.axis_index("c")
        zeros = jnp.zeros((_LANES,), jnp.int32)

        def zero_body(i, carry):
            hist0_v[pl.ds(i * _LANES, _LANES)] = zeros
            hist1_v[pl.ds(i * _LANES, _LANES)] = zeros
            return carry

        lax.fori_loop(0, _K // _LANES, zero_body, 0)

        pltpu.sync_copy(idx_hbm.at[pl.ds(wid * _ROWS_W, _ROWS_W)], idx_v)

        ones = jnp.ones((_LANES,), jnp.int32)
        hists = [hist0_v, hist1_v]

        @plsc.parallel_loop(0, _ROWS_W, unroll=4)
        def _(r):
            for j in range(_VECS_ROW):
                v = idx_v[r, pl.ds(j * _LANES, _LANES)]
                plsc.addupdate_scatter(hists[j % _NH], [v], ones)

        pltpu.sync_copy(hist0_v, out_hbm.at[wid * _NH])
        pltpu.sync_copy(hist1_v, out_hbm.at[wid * _NH + 1])

    return hist_kernel(quant_info)


def _tc_entropy(parts):
    def body(parts_ref, out_ref):
        x = parts_ref[...].astype(jnp.float32)
        counts = jnp.sum(x, axis=0, keepdims=True)
        total = jnp.sum(counts)
        p = counts / total
        ent = -jnp.sum(p * jnp.log(p + _EPS))
        out_ref[0, 0] = jnp.exp(ent)

    return pl.pallas_call(
        body,
        out_shape=jax.ShapeDtypeStruct((1, 1), jnp.float32),
        out_specs=pl.BlockSpec(memory_space=pltpu.SMEM),
    )(parts)


def kernel(quant_info, iteration):
    parts = _sc_hist(quant_info)
    perp = _tc_entropy(parts)
    return perp[0, 0]


# speedup vs baseline: 1.1318x; 1.1318x over previous
import functools

import jax
import jax.numpy as jnp
import numpy as np
from jax import lax
from jax.experimental import pallas as pl
from jax.experimental.pallas import tpu as pltpu
from jax.experimental.pallas import tpu_sc as plsc

_EPS = float(np.finfo(np.float32).eps)
_K = 8192
_R = 1024
_C = 1024
_NC = 2
_NS = 16
_NW = _NC * _NS
_ROWS_W = _R // _NW
_LANES = 16
_NH = 2
_VECS_ROW = _C // _LANES


def _sc_hist(quant_info):
    mesh = plsc.VectorSubcoreMesh(core_axis_name="c", subcore_axis_name="s")

    @functools.partial(
        pl.kernel,
        out_type=jax.ShapeDtypeStruct((_NW * _NH, _K), jnp.int32),
        mesh=mesh,
        scratch_types=[
            pltpu.VMEM((_ROWS_W, _C), jnp.int32),
            pltpu.VMEM((_K,), jnp.int32),
            pltpu.VMEM((_K,), jnp.int32),
        ],
        compiler_params=pltpu.CompilerParams(
            needs_layout_passes=False, disable_bounds_checks=True),
    )
    def hist_kernel(idx_hbm, out_hbm, idx_v, hist0_v, hist1_v):
        wid = lax.axis_index("s") * _NC + lax.axis_index("c")
        zeros = jnp.zeros((_LANES,), jnp.int32)

        def zero_body(i, carry):
            hist0_v[pl.ds(i * _LANES, _LANES)] = zeros
            hist1_v[pl.ds(i * _LANES, _LANES)] = zeros
            return carry

        lax.fori_loop(0, _K // _LANES, zero_body, 0)

        pltpu.sync_copy(idx_hbm.at[pl.ds(wid * _ROWS_W, _ROWS_W)], idx_v)

        ones = jnp.ones((_LANES,), jnp.int32)
        hists = [hist0_v, hist1_v]

        @plsc.parallel_loop(0, _ROWS_W * _VECS_ROW, unroll=8)
        def _(i):
            r = i >> 6
            j = i & (_VECS_ROW - 1)
            v = idx_v[r, pl.ds(j * _LANES, _LANES)]
            plsc.addupdate_scatter(hists[0], [v], ones)

        pltpu.sync_copy(hist0_v, out_hbm.at[wid * _NH])
        pltpu.sync_copy(hist1_v, out_hbm.at[wid * _NH + 1])

    return hist_kernel(quant_info)


def _tc_entropy(parts):
    def body(parts_ref, out_ref):
        x = parts_ref[...].astype(jnp.float32)
        counts = jnp.sum(x, axis=0, keepdims=True)
        total = jnp.sum(counts)
        p = counts / total
        ent = -jnp.sum(p * jnp.log(p + _EPS))
        out_ref[0, 0] = jnp.exp(ent)

    return pl.pallas_call(
        body,
        out_shape=jax.ShapeDtypeStruct((1, 1), jnp.float32),
        out_specs=pl.BlockSpec(memory_space=pltpu.SMEM),
    )(parts)


def kernel(quant_info, iteration):
    parts = _sc_hist(quant_info)
    perp = _tc_entropy(parts)
    return perp[0, 0]


# speedup vs baseline: 1.1359x; 1.0037x over previous
import functools

import jax
import jax.numpy as jnp
import numpy as np
from jax import lax
from jax.experimental import pallas as pl
from jax.experimental.pallas import tpu as pltpu
from jax.experimental.pallas import tpu_sc as plsc

_EPS = float(np.finfo(np.float32).eps)
_K = 8192
_R = 1024
_C = 1024
_NC = 2
_NS = 16
_NW = _NC * _NS
_ROWS_W = _R // _NW
_LANES = 16
_NH = 2
_VECS_ROW = _C // _LANES


def _sc_hist(quant_info):
    mesh = plsc.VectorSubcoreMesh(core_axis_name="c", subcore_axis_name="s")

    @functools.partial(
        pl.kernel,
        out_type=jax.ShapeDtypeStruct((_NW * _NH, _K), jnp.int32),
        mesh=mesh,
        scratch_types=[
            pltpu.VMEM((_ROWS_W, _C), jnp.int32),
            pltpu.VMEM((_K,), jnp.int32),
            pltpu.VMEM((_K,), jnp.int32),
        ],
        compiler_params=pltpu.CompilerParams(
            needs_layout_passes=False, disable_bounds_checks=True),
    )
    def hist_kernel(idx_hbm, out_hbm, idx_v, hist0_v, hist1_v):
        wid = lax.axis_index("s") * _NC + lax.axis_index("c")
        zeros = jnp.zeros((_LANES,), jnp.int32)

        def zero_body(i, carry):
            hist0_v[pl.ds(i * _LANES, _LANES)] = zeros
            hist1_v[pl.ds(i * _LANES, _LANES)] = zeros
            return carry

        lax.fori_loop(0, _K // _LANES, zero_body, 0)

        pltpu.sync_copy(idx_hbm.at[pl.ds(wid * _ROWS_W, _ROWS_W)], idx_v)

        ones = jnp.ones((_LANES,), jnp.int32)
        hists = [hist0_v, hist1_v]

        @plsc.parallel_loop(0, _ROWS_W * _VECS_ROW, unroll=16)
        def _(i):
            r = i >> 6
            j = i & (_VECS_ROW - 1)
            v = idx_v[r, pl.ds(j * _LANES, _LANES)]
            plsc.addupdate_scatter(hists[0], [v], ones)

        pltpu.sync_copy(hist0_v, out_hbm.at[wid * _NH])
        pltpu.sync_copy(hist1_v, out_hbm.at[wid * _NH + 1])

    return hist_kernel(quant_info)


def _tc_entropy(parts):
    def body(parts_ref, out_ref):
        x = parts_ref[...].astype(jnp.float32)
        counts = jnp.sum(x, axis=0, keepdims=True)
        total = jnp.sum(counts)
        p = counts / total
        ent = -jnp.sum(p * jnp.log(p + _EPS))
        out_ref[0, 0] = jnp.exp(ent)

    return pl.pallas_call(
        body,
        out_shape=jax.ShapeDtypeStruct((1, 1), jnp.float32),
        out_specs=pl.BlockSpec(memory_space=pltpu.SMEM),
    )(parts)


def kernel(quant_info, iteration):
    parts = _sc_hist(quant_info)
    perp = _tc_entropy(parts)
    return perp[0, 0]


# speedup vs baseline: 1.1659x; 1.0264x over previous
import functools

import jax
import jax.numpy as jnp
import numpy as np
from jax import lax
from jax.experimental import pallas as pl
from jax.experimental.pallas import tpu as pltpu
from jax.experimental.pallas import tpu_sc as plsc

_EPS = float(np.finfo(np.float32).eps)
_K = 8192
_R = 1024
_C = 1024
_NC = 2
_NS = 16
_NW = _NC * _NS
_ROWS_W = _R // _NW
_LANES = 16
_VECS_ROW = _C // _LANES


def _sc_hist(quant_info):
    mesh = plsc.VectorSubcoreMesh(core_axis_name="c", subcore_axis_name="s")

    @functools.partial(
        pl.kernel,
        out_type=jax.ShapeDtypeStruct((_NW, _K), jnp.int32),
        mesh=mesh,
        scratch_types=[
            pltpu.VMEM((_ROWS_W, _C), jnp.int32),
            pltpu.VMEM((_K,), jnp.int32),
        ],
        compiler_params=pltpu.CompilerParams(
            needs_layout_passes=False, disable_bounds_checks=True),
    )
    def hist_kernel(idx_hbm, out_hbm, idx_v, hist_v):
        wid = lax.axis_index("s") * _NC + lax.axis_index("c")
        zeros = jnp.zeros((_LANES,), jnp.int32)

        def zero_body(i, carry):
            hist_v[pl.ds(i * _LANES, _LANES)] = zeros
            return carry

        lax.fori_loop(0, _K // _LANES, zero_body, 0)

        pltpu.sync_copy(idx_hbm.at[pl.ds(wid * _ROWS_W, _ROWS_W)], idx_v)

        ones = jnp.ones((_LANES,), jnp.int32)

        @plsc.parallel_loop(0, _ROWS_W * _VECS_ROW, unroll=16)
        def _(i):
            r = i >> 6
            j = i & (_VECS_ROW - 1)
            v = idx_v[r, pl.ds(j * _LANES, _LANES)]
            plsc.addupdate_scatter(hist_v, [v], ones)

        pltpu.sync_copy(hist_v, out_hbm.at[wid])

    return hist_kernel(quant_info)


def _tc_entropy(parts):
    def body(parts_ref, out_ref):
        x = parts_ref[...].astype(jnp.float32)
        counts = jnp.sum(x, axis=0, keepdims=True)
        total = jnp.sum(counts)
        p = counts / total
        ent = -jnp.sum(p * jnp.log(p + _EPS))
        out_ref[0, 0] = jnp.exp(ent)

    return pl.pallas_call(
        body,
        out_shape=jax.ShapeDtypeStruct((1, 1), jnp.float32),
        out_specs=pl.BlockSpec(memory_space=pltpu.SMEM),
    )(parts)


def kernel(quant_info, iteration):
    parts = _sc_hist(quant_info)
    perp = _tc_entropy(parts)
    return perp[0, 0]


# speedup vs baseline: 1.2497x; 1.0719x over previous
import functools

import jax
import jax.numpy as jnp
import numpy as np
from jax import lax
from jax.experimental import pallas as pl
from jax.experimental.pallas import tpu as pltpu
from jax.experimental.pallas import tpu_sc as plsc

_EPS = float(np.finfo(np.float32).eps)
_K = 8192
_R = 1024
_C = 1024
_NC = 2
_NS = 16
_NW = _NC * _NS
_ROWS_W = _R // _NW
_LANES = 16
_VECS_ROW = _C // _LANES


def _sc_hist(quant_info):
    mesh = plsc.VectorSubcoreMesh(core_axis_name="c", subcore_axis_name="s")

    @functools.partial(
        pl.kernel,
        out_type=jax.ShapeDtypeStruct((_NW, _K), jnp.int32),
        mesh=mesh,
        scratch_types=[
            pltpu.VMEM((_ROWS_W, _C), jnp.int32),
            pltpu.VMEM((_K,), jnp.int32),
            [pltpu.SemaphoreType.DMA] * 4,
        ],
        compiler_params=pltpu.CompilerParams(
            needs_layout_passes=False, disable_bounds_checks=True),
    )
    def hist_kernel(idx_hbm, out_hbm, idx_v, hist_v, sems):
        wid = lax.axis_index("s") * _NC + lax.axis_index("c")
        base = wid * _ROWS_W
        chunk = _ROWS_W // 4

        copies = [
            pltpu.async_copy(
                idx_hbm.at[pl.ds(base + c * chunk, chunk)],
                idx_v.at[pl.ds(c * chunk, chunk)],
                sems[c],
            )
            for c in range(4)
        ]

        zeros = jnp.zeros((_LANES,), jnp.int32)

        @plsc.parallel_loop(0, _K // _LANES, unroll=8)
        def _(i):
            hist_v[pl.ds(i * _LANES, _LANES)] = zeros

        ones = jnp.ones((_LANES,), jnp.int32)
        vecs_chunk = chunk * _VECS_ROW

        for c in range(4):
            copies[c].wait()

            @plsc.parallel_loop(c * vecs_chunk, (c + 1) * vecs_chunk, unroll=16)
            def _(i):
                r = i >> 6
                j = i & (_VECS_ROW - 1)
                v = idx_v[r, pl.ds(j * _LANES, _LANES)]
                plsc.addupdate_scatter(hist_v, [v], ones)

        pltpu.sync_copy(hist_v, out_hbm.at[wid])

    return hist_kernel(quant_info)


def _tc_entropy(parts):
    def body(parts_ref, out_ref):
        x = parts_ref[...].astype(jnp.float32)
        counts = jnp.sum(x, axis=0, keepdims=True)
        total = jnp.sum(counts)
        p = counts / total
        ent = -jnp.sum(p * jnp.log(p + _EPS))
        out_ref[0, 0] = jnp.exp(ent)

    return pl.pallas_call(
        body,
        out_shape=jax.ShapeDtypeStruct((1, 1), jnp.float32),
        out_specs=pl.BlockSpec(memory_space=pltpu.SMEM),
    )(parts)


def kernel(quant_info, iteration):
    parts = _sc_hist(quant_info)
    perp = _tc_entropy(parts)
    return perp[0, 0]
